# baseline (speedup 1.0000x reference)
import functools

import jax
import jax.numpy as jnp
from jax import lax
from jax.experimental import pallas as pl
from jax.experimental.pallas import tpu as pltpu
from jax.experimental.pallas import tpu_sc as plsc

_R_FRAC = 0.01
_NB = 8192
_SHIFT = 19


def _sc_hist(x1d, *, Bs, N):
    NW = 32
    RW = Bs // NW
    CH = 10000
    nch = N // CH
    mesh = plsc.VectorSubcoreMesh(core_axis_name="c", subcore_axis_name="s")

    @functools.partial(
        pl.kernel, mesh=mesh,
        compiler_params=pltpu.CompilerParams(needs_layout_passes=False),
        out_type=[jax.ShapeDtypeStruct((Bs * 4 * _NB,), jnp.int32),
                  jax.ShapeDtypeStruct((Bs * 4 * _NB,), jnp.float32)],
        scratch_types=[pltpu.VMEM((CH,), jnp.float32),
                       pltpu.VMEM((CH,), jnp.float32),
                       pltpu.VMEM((4 * _NB,), jnp.int32),
                       pltpu.VMEM((4 * _NB,), jnp.float32),
                       pltpu.SemaphoreType.DMA,
                       pltpu.SemaphoreType.DMA],
    )
    def hist_kernel(x_hbm, cnt_hbm, esum_hbm,
                    buf0, buf1, hcnt, hesum, sem0, sem1):
        wid = lax.axis_index("s") * 2 + lax.axis_index("c")
        base = wid * RW

        bufs = (buf0, buf1)
        sems = (sem0, sem1)

        def row_body(rr, carry):
            r = base + rr

            def zb(i, c):
                for u in range(8):
                    j = i * 8 + u
                    hcnt[pl.ds(j * 16, 16)] = jnp.zeros((16,), jnp.int32)
                    hesum[pl.ds(j * 16, 16)] = jnp.zeros((16,), jnp.float32)
                return c
            lax.fori_loop(0, 4 * _NB // (16 * 8), zb, 0)

            rbase = pl.multiple_of(r * N, 8)
            hprev = pltpu.make_async_copy(
                x_hbm.at[pl.ds(rbase, CH)], buf0, sem0)
            hprev.start()
            for c in range(nch):
                hcur = hprev
                if c + 1 < nch:
                    hprev = pltpu.make_async_copy(
                        x_hbm.at[pl.ds(pl.multiple_of(r * N + (c + 1) * CH, 8), CH)],
                        bufs[(c + 1) % 2], sems[(c + 1) % 2])
                    hprev.start()
                hcur.wait()
                b = bufs[c % 2]

                lane4 = (lax.iota(jnp.int32, 16) & jnp.int32(3)) * jnp.int32(_NB)

                def pb(i, cc):
                    for u in range(25):
                        j = i * 25 + u
                        v = b[pl.ds(j * 16, 16)]
                        bi = lax.bitcast_convert_type(v, jnp.int32)
                        key = jnp.where(bi >= 0, bi,
                                        bi ^ jnp.int32(0x7FFFFFFF))
                        bin_ = (lax.shift_right_arithmetic(key, _SHIFT)
                                + jnp.int32(_NB // 2)) + lane4
                        plsc.addupdate_scatter(hcnt, [bin_],
                                               jnp.ones((16,), jnp.int32))
                        plsc.addupdate_scatter(hesum, [bin_],
                                               jnp.exp(v * 10.0))
                    return cc
                lax.fori_loop(0, CH // (16 * 25), pb, 0)

            hb = pl.multiple_of(r * 4 * _NB, 8)
            pltpu.sync_copy(hcnt, cnt_hbm.at[pl.ds(hb, 4 * _NB)])
            pltpu.sync_copy(hesum, esum_hbm.at[pl.ds(hb, 4 * _NB)])
            return carry

        lax.fori_loop(0, RW, row_body, 0)

    return hist_kernel(x1d)


def _consume_block(cnt_ref, esum_ref, pos_ref, out_ref, *, K):
    cnt4 = cnt_ref[...].astype(jnp.float32)
    esum4 = esum_ref[...]
    cnt = (cnt4[:, :_NB] + cnt4[:, _NB:2 * _NB]
           + cnt4[:, 2 * _NB:3 * _NB] + cnt4[:, 3 * _NB:])
    esum = (esum4[:, :_NB] + esum4[:, _NB:2 * _NB]
            + esum4[:, 2 * _NB:3 * _NB] + esum4[:, 3 * _NB:])
    pos = pos_ref[...]
    binid = lax.broadcasted_iota(jnp.int32, cnt.shape, 1)

    pb_ = lax.bitcast_convert_type(pos, jnp.int32)
    pkey = jnp.where(pb_ >= 0, pb_, pb_ ^ jnp.int32(0x7FFFFFFF))
    pbin = lax.shift_right_arithmetic(pkey, _SHIFT) + jnp.int32(_NB // 2)

    Kf = jnp.float32(K)

    def step(i, lo):
        cand = lo + (jnp.int32(1) << (12 - i))
        Cc = (jnp.sum(jnp.where(binid >= cand, cnt, 0.0),
                      axis=1, keepdims=True)
              - (pbin >= cand).astype(jnp.float32))
        return jnp.where(Cc >= Kf, cand, lo)

    bk = lax.fori_loop(0, 13, step, jnp.zeros_like(pbin))

    e10p = jnp.exp(10.0 * pos)
    p_above = (pbin > bk).astype(jnp.float32)
    p_at = (pbin == bk).astype(jnp.float32)

    above = binid > bk
    at = binid == bk
    c_hi = (jnp.sum(jnp.where(above, cnt, 0.0), axis=1, keepdims=True)
            - p_above)
    S_hi = jnp.sum(jnp.where(above, esum, 0.0), axis=1, keepdims=True)
    c_b = (jnp.sum(jnp.where(at, cnt, 0.0), axis=1, keepdims=True)
           - p_at)
    S_b = (jnp.sum(jnp.where(at, esum, 0.0), axis=1, keepdims=True)
           - p_at * e10p)
    need = jnp.clip(Kf - c_hi, 0.0, c_b)
    S = S_hi + need * S_b / jnp.maximum(c_b, 1.0) + (1.0 - p_above) * e10p
    out_ref[...] = jnp.log(S) - 10.0 * pos


_V_STEPS = 10


def _mmcl_block(x_ref, t_ref, out_ref, *, K, N):
    x = x_ref[...]
    t = t_ref[...]
    col = jax.lax.broadcasted_iota(jnp.int32, x.shape, 1)
    is_t = col == t
    valid_neg = (col < N) & jnp.logical_not(is_t)
    pos = jnp.sum(jnp.where(is_t, x, 0.0), axis=1, keepdims=True)
    xn = jnp.where(valid_neg, x, -jnp.inf)
    mneg = jnp.max(xn, axis=1, keepdims=True)
    m = jnp.maximum(mneg, pos)
    mn = jnp.min(jnp.where(valid_neg, x, jnp.inf), axis=1, keepdims=True)

    K_ = jnp.int32(K)
    lo0 = mn
    hi0 = mneg + jnp.maximum(jnp.abs(mneg) * 9.8e-4, 1e-30)

    def body(i, carry):
        lo, hi = carry
        mid = 0.5 * (lo + hi)
        cnt = jnp.sum((xn >= mid).astype(jnp.int32), axis=1, keepdims=True)
        ok = cnt >= K_
        return jnp.where(ok, mid, lo), jnp.where(ok, hi, mid)

    lo, hi = jax.lax.fori_loop(0, _V_STEPS, body, (lo0, hi0))

    e = jnp.exp(10.0 * (xn - m))
    ge_hi = xn >= hi
    in_b = (xn >= lo) & jnp.logical_not(ge_hi)
    c_hi = jnp.sum(ge_hi.astype(jnp.int32), axis=1, keepdims=True)
    S_hi = jnp.sum(jnp.where(ge_hi, e, 0.0), axis=1, keepdims=True)
    c_b = jnp.sum(in_b.astype(jnp.int32), axis=1, keepdims=True)
    S_b = jnp.sum(jnp.where(in_b, e, 0.0), axis=1, keepdims=True)
    c_b = jnp.maximum(c_b, 1)
    S = (S_hi
         + (K_ - c_hi).astype(jnp.float32) * S_b / c_b.astype(jnp.float32)
         + jnp.exp(10.0 * (pos - m)))
    out_ref[...] = 10.0 * (m - pos) + jnp.log(S)


def _losses_bisect(logits, t2, *, K, N):
    Bs = logits.shape[0]
    R = 16
    return pl.pallas_call(
        functools.partial(_mmcl_block, K=K, N=N),
        grid=(Bs // R,),
        in_specs=[
            pl.BlockSpec((R, N), lambda i: (i, 0)),
            pl.BlockSpec((R, 1), lambda i: (i, 0)),
        ],
        out_specs=pl.BlockSpec((R, 1), lambda i: (i, 0)),
        out_shape=jax.ShapeDtypeStruct((Bs, 1), jnp.float32),
    )(logits, t2)


def _losses_shard(x, t2, *, K, N):
    B0 = x.shape[0]
    s = B0 // 8
    x_tc, t_tc = x[s:], t2[s:]
    x, t2 = x[:s], t2[:s]
    losses_tc = _losses_bisect(x_tc, t_tc, K=K, N=N)
    Bs = x.shape[0]
    x1d = x.reshape(Bs * N)
    cnt, esum = _sc_hist(x1d, Bs=Bs, N=N)
    cnt = cnt.reshape(Bs, 4 * _NB)
    esum = esum.reshape(Bs, 4 * _NB)
    pos = jnp.take_along_axis(x, t2, axis=1)
    Rb = 16
    losses_sc = pl.pallas_call(
        functools.partial(_consume_block, K=K),
        grid=(Bs // Rb,),
        in_specs=[
            pl.BlockSpec((Rb, 4 * _NB), lambda i: (i, 0)),
            pl.BlockSpec((Rb, 4 * _NB), lambda i: (i, 0)),
            pl.BlockSpec((Rb, 1), lambda i: (i, 0)),
        ],
        out_specs=pl.BlockSpec((Rb, 1), lambda i: (i, 0)),
        out_shape=jax.ShapeDtypeStruct((Bs, 1), jnp.float32),
    )(cnt, esum, pos)
    return jnp.concatenate([losses_sc, losses_tc], axis=0)


def kernel(logits, targets):
    B, N = logits.shape
    K = int(_R_FRAC * (N - 1))
    t2 = targets.reshape(B, 1).astype(jnp.int32)
    f = functools.partial(_losses_shard, K=K, N=N)

    devs = jax.devices()
    ndev = 2 if (len(devs) >= 2 and B % 64 == 0) else 1
    if ndev > 1:
        import numpy as np
        from jax.sharding import Mesh, PartitionSpec as P
        mesh = Mesh(np.asarray(devs[:ndev]), ("b",))
        f = jax.shard_map(f, mesh=mesh,
                          in_specs=(P("b", None), P("b", None)),
                          out_specs=P("b", None), check_vma=False)
    return jnp.mean(f(logits, t2))


# speedup vs baseline: 1.0626x; 1.0626x over previous
import functools

import jax
import jax.numpy as jnp
from jax import lax
from jax.experimental import pallas as pl
from jax.experimental.pallas import tpu as pltpu
from jax.experimental.pallas import tpu_sc as plsc

_R_FRAC = 0.01
_NB = 8192
_SHIFT = 19


def _sc_hist(x1d, *, Bs, N):
    NW = 32
    RW = Bs // NW
    CH = 10000
    nch = N // CH
    mesh = plsc.VectorSubcoreMesh(core_axis_name="c", subcore_axis_name="s")

    @functools.partial(
        pl.kernel, mesh=mesh,
        compiler_params=pltpu.CompilerParams(needs_layout_passes=False),
        out_type=[jax.ShapeDtypeStruct((Bs * 4 * _NB,), jnp.int32),
                  jax.ShapeDtypeStruct((Bs * 4 * _NB,), jnp.float32)],
        scratch_types=[pltpu.VMEM((CH,), jnp.float32),
                       pltpu.VMEM((CH,), jnp.float32),
                       pltpu.VMEM((4 * _NB,), jnp.int32),
                       pltpu.VMEM((4 * _NB,), jnp.float32),
                       pltpu.SemaphoreType.DMA,
                       pltpu.SemaphoreType.DMA],
    )
    def hist_kernel(x_hbm, cnt_hbm, esum_hbm,
                    buf0, buf1, hcnt, hesum, sem0, sem1):
        wid = lax.axis_index("s") * 2 + lax.axis_index("c")
        base = wid * RW

        bufs = (buf0, buf1)
        sems = (sem0, sem1)

        def row_body(rr, carry):
            r = base + rr

            def zb(i, c):
                for u in range(8):
                    j = i * 8 + u
                    hcnt[pl.ds(j * 16, 16)] = jnp.zeros((16,), jnp.int32)
                    hesum[pl.ds(j * 16, 16)] = jnp.zeros((16,), jnp.float32)
                return c
            lax.fori_loop(0, 4 * _NB // (16 * 8), zb, 0)

            rbase = pl.multiple_of(r * N, 8)
            hprev = pltpu.make_async_copy(
                x_hbm.at[pl.ds(rbase, CH)], buf0, sem0)
            hprev.start()
            for c in range(nch):
                hcur = hprev
                if c + 1 < nch:
                    hprev = pltpu.make_async_copy(
                        x_hbm.at[pl.ds(pl.multiple_of(r * N + (c + 1) * CH, 8), CH)],
                        bufs[(c + 1) % 2], sems[(c + 1) % 2])
                    hprev.start()
                hcur.wait()
                b = bufs[c % 2]

                lane4 = (lax.iota(jnp.int32, 16) & jnp.int32(3)) * jnp.int32(_NB)

                def pb(i, cc):
                    for u in range(25):
                        j = i * 25 + u
                        v = b[pl.ds(j * 16, 16)]
                        bi = lax.bitcast_convert_type(v, jnp.int32)
                        key = jnp.where(bi >= 0, bi,
                                        bi ^ jnp.int32(0x7FFFFFFF))
                        bin_ = (lax.shift_right_arithmetic(key, _SHIFT)
                                + jnp.int32(_NB // 2)) + lane4
                        plsc.addupdate_scatter(hcnt, [bin_],
                                               jnp.ones((16,), jnp.int32))
                        plsc.addupdate_scatter(hesum, [bin_],
                                               jnp.exp(v * 10.0))
                    return cc
                lax.fori_loop(0, CH // (16 * 25), pb, 0)

            hb = pl.multiple_of(r * 4 * _NB, 8)
            pltpu.sync_copy(hcnt, cnt_hbm.at[pl.ds(hb, 4 * _NB)])
            pltpu.sync_copy(hesum, esum_hbm.at[pl.ds(hb, 4 * _NB)])
            return carry

        lax.fori_loop(0, RW, row_body, 0)

    return hist_kernel(x1d)


def _consume_block(cnt_ref, esum_ref, pos_ref, out_ref, *, K):
    cnt4 = cnt_ref[...].astype(jnp.float32)
    esum4 = esum_ref[...]
    cnt = (cnt4[:, :_NB] + cnt4[:, _NB:2 * _NB]
           + cnt4[:, 2 * _NB:3 * _NB] + cnt4[:, 3 * _NB:])
    esum = (esum4[:, :_NB] + esum4[:, _NB:2 * _NB]
            + esum4[:, 2 * _NB:3 * _NB] + esum4[:, 3 * _NB:])
    pos = pos_ref[...]
    binid = lax.broadcasted_iota(jnp.int32, cnt.shape, 1)

    pb_ = lax.bitcast_convert_type(pos, jnp.int32)
    pkey = jnp.where(pb_ >= 0, pb_, pb_ ^ jnp.int32(0x7FFFFFFF))
    pbin = lax.shift_right_arithmetic(pkey, _SHIFT) + jnp.int32(_NB // 2)

    Kf = jnp.float32(K)

    def step(i, lo):
        cand = lo + (jnp.int32(1) << (12 - i))
        Cc = (jnp.sum(jnp.where(binid >= cand, cnt, 0.0),
                      axis=1, keepdims=True)
              - (pbin >= cand).astype(jnp.float32))
        return jnp.where(Cc >= Kf, cand, lo)

    bk = lax.fori_loop(0, 13, step, jnp.zeros_like(pbin))

    e10p = jnp.exp(10.0 * pos)
    p_above = (pbin > bk).astype(jnp.float32)
    p_at = (pbin == bk).astype(jnp.float32)

    above = binid > bk
    at = binid == bk
    c_hi = (jnp.sum(jnp.where(above, cnt, 0.0), axis=1, keepdims=True)
            - p_above)
    S_hi = jnp.sum(jnp.where(above, esum, 0.0), axis=1, keepdims=True)
    c_b = (jnp.sum(jnp.where(at, cnt, 0.0), axis=1, keepdims=True)
           - p_at)
    S_b = (jnp.sum(jnp.where(at, esum, 0.0), axis=1, keepdims=True)
           - p_at * e10p)
    need = jnp.clip(Kf - c_hi, 0.0, c_b)
    S = S_hi + need * S_b / jnp.maximum(c_b, 1.0) + (1.0 - p_above) * e10p
    out_ref[...] = jnp.log(S) - 10.0 * pos


_V_STEPS = 10


def _mmcl_block(x_ref, t_ref, out_ref, *, K, N):
    x = x_ref[...]
    t = t_ref[...]
    col = jax.lax.broadcasted_iota(jnp.int32, x.shape, 1)
    is_t = col == t
    valid_neg = (col < N) & jnp.logical_not(is_t)
    pos = jnp.sum(jnp.where(is_t, x, 0.0), axis=1, keepdims=True)
    xn = jnp.where(valid_neg, x, -jnp.inf)
    mneg = jnp.max(xn, axis=1, keepdims=True)
    m = jnp.maximum(mneg, pos)
    mn = jnp.min(jnp.where(valid_neg, x, jnp.inf), axis=1, keepdims=True)

    K_ = jnp.int32(K)
    lo0 = mn
    hi0 = mneg + jnp.maximum(jnp.abs(mneg) * 9.8e-4, 1e-30)

    def body(i, carry):
        lo, hi = carry
        mid = 0.5 * (lo + hi)
        cnt = jnp.sum((xn >= mid).astype(jnp.int32), axis=1, keepdims=True)
        ok = cnt >= K_
        return jnp.where(ok, mid, lo), jnp.where(ok, hi, mid)

    lo, hi = jax.lax.fori_loop(0, _V_STEPS, body, (lo0, hi0))

    e = jnp.exp(10.0 * (xn - m))
    ge_hi = xn >= hi
    in_b = (xn >= lo) & jnp.logical_not(ge_hi)
    c_hi = jnp.sum(ge_hi.astype(jnp.int32), axis=1, keepdims=True)
    S_hi = jnp.sum(jnp.where(ge_hi, e, 0.0), axis=1, keepdims=True)
    c_b = jnp.sum(in_b.astype(jnp.int32), axis=1, keepdims=True)
    S_b = jnp.sum(jnp.where(in_b, e, 0.0), axis=1, keepdims=True)
    c_b = jnp.maximum(c_b, 1)
    S = (S_hi
         + (K_ - c_hi).astype(jnp.float32) * S_b / c_b.astype(jnp.float32)
         + jnp.exp(10.0 * (pos - m)))
    out_ref[...] = 10.0 * (m - pos) + jnp.log(S)


def _losses_bisect(logits, t2, *, K, N):
    Bs = logits.shape[0]
    R = 16
    return pl.pallas_call(
        functools.partial(_mmcl_block, K=K, N=N),
        grid=(Bs // R,),
        in_specs=[
            pl.BlockSpec((R, N), lambda i: (i, 0)),
            pl.BlockSpec((R, 1), lambda i: (i, 0)),
        ],
        out_specs=pl.BlockSpec((R, 1), lambda i: (i, 0)),
        out_shape=jax.ShapeDtypeStruct((Bs, 1), jnp.float32),
    )(logits, t2)


def _losses_shard(x, t2, *, K, N):
    B0 = x.shape[0]
    s = B0 // 4
    x_tc, t_tc = x[s:], t2[s:]
    x, t2 = x[:s], t2[:s]
    losses_tc = _losses_bisect(x_tc, t_tc, K=K, N=N)
    Bs = x.shape[0]
    x1d = x.reshape(Bs * N)
    cnt, esum = _sc_hist(x1d, Bs=Bs, N=N)
    cnt = cnt.reshape(Bs, 4 * _NB)
    esum = esum.reshape(Bs, 4 * _NB)
    pos = jnp.take_along_axis(x, t2, axis=1)
    Rb = 16
    losses_sc = pl.pallas_call(
        functools.partial(_consume_block, K=K),
        grid=(Bs // Rb,),
        in_specs=[
            pl.BlockSpec((Rb, 4 * _NB), lambda i: (i, 0)),
            pl.BlockSpec((Rb, 4 * _NB), lambda i: (i, 0)),
            pl.BlockSpec((Rb, 1), lambda i: (i, 0)),
        ],
        out_specs=pl.BlockSpec((Rb, 1), lambda i: (i, 0)),
        out_shape=jax.ShapeDtypeStruct((Bs, 1), jnp.float32),
    )(cnt, esum, pos)
    return jnp.concatenate([losses_sc, losses_tc], axis=0)


def kernel(logits, targets):
    B, N = logits.shape
    K = int(_R_FRAC * (N - 1))
    t2 = targets.reshape(B, 1).astype(jnp.int32)
    f = functools.partial(_losses_shard, K=K, N=N)

    devs = jax.devices()
    ndev = 2 if (len(devs) >= 2 and B % 64 == 0) else 1
    if ndev > 1:
        import numpy as np
        from jax.sharding import Mesh, PartitionSpec as P
        mesh = Mesh(np.asarray(devs[:ndev]), ("b",))
        f = jax.shard_map(f, mesh=mesh,
                          in_specs=(P("b", None), P("b", None)),
                          out_specs=P("b", None), check_vma=False)
    return jnp.mean(f(logits, t2))
